# baseline (speedup 1.0000x reference)
import functools

import jax
import jax.numpy as jnp
from jax import lax
from jax.experimental import pallas as pl
from jax.experimental.pallas import tpu as pltpu
from jax.experimental.pallas import tpu_sc as plsc

K = 1024
F = 256
N = 4096

NB = 2048
NBLK = N // NB

NC = 2
NS = 16
NW = NC * NS
PW = N // NW
RW = K // NS
CW = 16


def _assign_body(c_ref, x_ref, nearest_ref, xt_ref, ct_ref, colsum_ref):
    C = c_ref[...]
    Xb = x_ref[...]
    xt_ref[...] = Xb.T.reshape(1, F, NB)
    c2 = jnp.sum(C * C, axis=1, keepdims=True)

    @pl.when(pl.program_id(0) == 0)
    def _():
        ct_ref[...] = C.T
        X1 = Xb[0:K, :]
        s_c = jnp.sum(C, axis=0, keepdims=True)
        x2 = jnp.sum(X1 * X1, axis=1, keepdims=True)
        xs = lax.dot_general(X1, s_c, (((1,), (1,)), ((), ())),
                             preferred_element_type=jnp.float32)
        colsum_ref[...] = jnp.sum(c2) + jnp.float32(K) * x2 - 2.0 * xs
    prod = lax.dot_general(C, Xb, (((1,), (1,)), ((), ())),
                           preferred_element_type=jnp.float32)
    d = c2 - 2.0 * prod
    mn = jnp.min(d, axis=0, keepdims=True)
    kio = lax.broadcasted_iota(jnp.int32, d.shape, 0)
    near = jnp.min(jnp.where(d == mn, kio, K), axis=0)
    nearest_ref[...] = near.reshape(1, 1, NB)


_assign = pl.pallas_call(
    _assign_body,
    grid=(NBLK,),
    in_specs=[
        pl.BlockSpec((K, F), lambda i: (0, 0)),
        pl.BlockSpec((NB, F), lambda i: (i, 0)),
    ],
    out_specs=[
        pl.BlockSpec((1, 1, NB), lambda i: (i, 0, 0)),
        pl.BlockSpec((1, F, NB), lambda i: (i, 0, 0)),
        pl.BlockSpec((F, K), lambda i: (0, 0)),
        pl.BlockSpec((K, 1), lambda i: (0, 0)),
    ],
    out_shape=[
        jax.ShapeDtypeStruct((NBLK, 1, NB), jnp.int32),
        jax.ShapeDtypeStruct((NBLK, F, NB), jnp.float32),
        jax.ShapeDtypeStruct((F, K), jnp.float32),
        jax.ShapeDtypeStruct((K, 1), jnp.float32),
    ],
)


NP = N // NC
BPC = NBLK // NC


def _segment_body(xt_hbm, idx_hbm,
                  sums_hbm, cnts_hbm,
                  idx_v, xtv, acc_v, cacc_v, sem):
    c = lax.axis_index("c")
    s = lax.axis_index("s")
    cps = [
        pltpu.make_async_copy(idx_hbm.at[pl.ds(c * NP, NP)], idx_v, sem),
        pltpu.make_async_copy(
            xt_hbm.at[pl.ds(c * BPC, BPC), pl.ds(s * CW, CW), :], xtv, sem),
    ]
    for cp in cps:
        cp.start()

    zeros16 = jnp.zeros((16,), jnp.float32)
    for f in range(CW):
        @plsc.parallel_loop(0, K // 16, 1, unroll=8)
        def zbody(i):
            acc_v[f, pl.ds(i * 16, 16)] = zeros16

    @plsc.parallel_loop(0, K // 16, 1, unroll=8)
    def zcbody(i):
        cacc_v[pl.ds(i * 16, 16)] = zeros16

    for cp in cps:
        cp.wait()

    for b in range(BPC):
        def sums_step(j, carry):
            rows0 = idx_v[pl.ds(b * NB + j * 32, 16)]
            rows1 = idx_v[pl.ds(b * NB + j * 32 + 16, 16)]

            @plsc.parallel_loop(0, CW, 1, unroll=CW)
            def fbody(f):
                fidx = jnp.full((16,), f, jnp.int32)
                vals0 = xtv[b, f, pl.ds(j * 32, 16)]
                vals1 = xtv[b, f, pl.ds(j * 32 + 16, 16)]
                plsc.addupdate_scatter(acc_v, [fidx, rows0], vals0)
                plsc.addupdate_scatter(acc_v, [fidx, rows1], vals1)

            return carry

        lax.fori_loop(0, NB // 32, sums_step, 0)

    ones = jnp.full((16,), 1.0, jnp.float32)

    def cnt_step(j, carry):
        rows = idx_v[pl.ds(j * 16, 16)]
        plsc.addupdate_scatter(cacc_v, [rows], ones)
        return carry

    lax.fori_loop(s * (PW // 16), (s + 1) * (PW // 16), cnt_step, 0)

    wps = [
        pltpu.make_async_copy(acc_v, sums_hbm.at[c, s], sem),
        pltpu.make_async_copy(cacc_v, cnts_hbm.at[c, s, 0], sem),
    ]
    for wp in wps:
        wp.start()
    for wp in wps:
        wp.wait()


@functools.cache
def _make_segment():
    return pl.kernel(
        _segment_body,
        mesh=plsc.VectorSubcoreMesh(core_axis_name="c", subcore_axis_name="s"),
        compiler_params=pltpu.CompilerParams(
            needs_layout_passes=False, disable_bounds_checks=True,
            use_tc_tiling_on_sc=True),
        out_type=[
            jax.ShapeDtypeStruct((NC, NS, CW, K), jnp.float32),
            jax.ShapeDtypeStruct((NC, NS, 1, K), jnp.float32),
        ],
        scratch_types=[
            pltpu.VMEM((NP,), jnp.int32),
            pltpu.VMEM((BPC, CW, NB), jnp.float32),
            pltpu.VMEM((CW, K), jnp.float32),
            pltpu.VMEM((K,), jnp.float32),
            pltpu.SemaphoreType.DMA,
        ],
    )


def _finalize_body(ct_ref, sums_ref, cnts_ref, colsum_ref, out1_ref, out2_ref):
    cnt = jnp.zeros((1, K), jnp.float32)
    for c in range(NC):
        for s in range(NS):
            cnt = cnt + cnts_ref[c, s]
    nonempty = cnt > 0.0
    safe = jnp.maximum(cnt, 1.0)
    deltas = jnp.zeros((1, K), jnp.float32)
    for s in range(NS):
        sg = sums_ref[0, s] + sums_ref[1, s]
        Cg = ct_ref[s * CW:(s + 1) * CW, :]
        ng = jnp.where(nonempty, sg / safe, Cg)
        deltas = deltas + jnp.sum(jnp.square(Cg - ng), axis=0, keepdims=True)
    sum_delta = jnp.sum(jnp.where(nonempty, deltas, 0.0))
    delta_k = jnp.sum(nonempty.astype(jnp.float32))
    avg = lax.dot_general(cnt, colsum_ref[...], (((1,), (0,)), ((), ())),
                          preferred_element_type=jnp.float32)[0, 0] / N
    out1_ref[0, 0] = sum_delta / delta_k
    out2_ref[0, 0] = avg


_finalize = pl.pallas_call(
    _finalize_body,
    grid=(1,),
    in_specs=[
        pl.BlockSpec((F, K), lambda i: (0, 0)),
        pl.BlockSpec((NC, NS, CW, K), lambda i: (0, 0, 0, 0)),
        pl.BlockSpec((NC, NS, 1, K), lambda i: (0, 0, 0, 0)),
        pl.BlockSpec((K, 1), lambda i: (0, 0)),
    ],
    out_specs=[
        pl.BlockSpec((1, 1), lambda i: (0, 0), memory_space=pltpu.SMEM),
        pl.BlockSpec((1, 1), lambda i: (0, 0), memory_space=pltpu.SMEM),
    ],
    out_shape=[
        jax.ShapeDtypeStruct((1, 1), jnp.float32),
        jax.ShapeDtypeStruct((1, 1), jnp.float32),
    ],
)


def kernel(x, centroids):
    nearest3, xt, ct, colsum = _assign(centroids, x)
    nearest = nearest3.reshape(N)
    sums_p, cnts_p = _make_segment()(xt, nearest)
    s1, s2 = _finalize(ct, sums_p, cnts_p, colsum)
    return (s1[0, 0], s2[0, 0])
